# Initial kernel scaffold; baseline (speedup 1.0000x reference)
import functools

import jax
import jax.numpy as jnp
from jax import lax
from jax.experimental import pallas as pl
from jax.experimental.pallas import tpu as pltpu
from jax.experimental.pallas import tpu_sc as plsc

_NC = 2
_NS = 16
_NW = _NC * _NS
_L = 16


@functools.lru_cache(maxsize=None)
def _make_sc_scores(N, V, D, K, T):
    n_per_w = N // _NW
    n_chunks = n_per_w // T
    n_groups = T // _L
    slots = 1 + K

    mesh = plsc.VectorSubcoreMesh(core_axis_name="c", subcore_axis_name="s")

    @functools.partial(
        pl.kernel,
        mesh=mesh,
        out_type=jax.ShapeDtypeStruct((8, N), jnp.float32),
        scratch_types=[
            pltpu.VMEM((V * D,), jnp.float32),
            pltpu.VMEM((T * D,), jnp.float32),
            pltpu.VMEM((slots, T), jnp.int32),
            pltpu.VMEM((8, T), jnp.float32),
        ],
    )
    def sc_scores(ctx_hbm, idx_hbm, w_hbm, out_hbm, w_v, c_v, i_v, o_v):
        wid = lax.axis_index("s") * _NC + lax.axis_index("c")
        base = wid * n_per_w
        pltpu.sync_copy(w_hbm, w_v)

        lane = lax.iota(jnp.int32, _L)

        def chunk_body(ci, _):
            cbase = base + ci * T
            pltpu.sync_copy(ctx_hbm.at[pl.ds(cbase * D, T * D)], c_v)
            pltpu.sync_copy(idx_hbm.at[:, pl.ds(cbase, T)], i_v)

            def group_body(g, _g):
                goff = g * _L
                cidx0 = (goff + lane) * D
                widx0 = [i_v[k, pl.ds(goff, _L)] * D for k in range(slots)]
                zeros = jnp.zeros((_L,), jnp.float32)
                carry0 = tuple([cidx0] + widx0 + [zeros] * slots)

                def d_body(_, carry):
                    cidx = carry[0]
                    widx = carry[1:1 + slots]
                    accs = carry[1 + slots:]
                    cv = plsc.load_gather(c_v, [cidx])
                    new_accs = []
                    for k in range(slots):
                        wv = plsc.load_gather(w_v, [widx[k]])
                        new_accs.append(accs[k] + wv * cv)
                    return tuple(
                        [cidx + 1]
                        + [widx[k] + 1 for k in range(slots)]
                        + new_accs
                    )

                out = lax.fori_loop(0, D, d_body, carry0, unroll=4)
                accs = out[1 + slots:]
                o_v[0, pl.ds(goff, _L)] = accs[0]
                for k in range(1, slots):
                    o_v[k, pl.ds(goff, _L)] = -accs[k]
                for k in range(slots, 8):
                    o_v[k, pl.ds(goff, _L)] = zeros
                return 0

            lax.fori_loop(0, n_groups, group_body, 0)
            pltpu.sync_copy(o_v, out_hbm.at[:, pl.ds(cbase, T)])
            return 0

        lax.fori_loop(0, n_chunks, chunk_body, 0)

    return sc_scores


def _tc_reduce_body(s_ref, o_ref):
    s = s_ref[...]
    ls = jax.nn.log_sigmoid(s)
    row = lax.broadcasted_iota(jnp.int32, s.shape, 0)
    contrib = jnp.where(row < 6, ls, 0.0)
    o_ref[0, 0] = -jnp.sum(contrib)


def kernel(X, context, W, neg_samples):
    B, S = X.shape
    V, D = W.shape
    K = neg_samples.shape[-1]
    N = B * S

    T = 640
    assert N % (_NW * T) == 0

    ctx_flat = context.reshape(N * D)
    w_flat = W.reshape(V * D)
    x_row = X.reshape(1, N).astype(jnp.int32)
    neg_rows = neg_samples.reshape(N, K).T.astype(jnp.int32)
    idxT = jnp.concatenate([x_row, neg_rows], axis=0)

    scores = _make_sc_scores(N, V, D, K, T)(ctx_flat, idxT, w_flat)

    loss = pl.pallas_call(
        _tc_reduce_body,
        out_shape=jax.ShapeDtypeStruct((1, 1), jnp.float32),
        out_specs=pl.BlockSpec(memory_space=pltpu.SMEM),
    )(scores)
    return loss[0, 0]


# speedup vs baseline: 3.9142x; 3.9142x over previous
import functools

import jax
import jax.numpy as jnp
from jax import lax
from jax.experimental import pallas as pl
from jax.experimental.pallas import tpu as pltpu
from jax.experimental.pallas import tpu_sc as plsc

_NC = 2
_NS = 16
_NW = _NC * _NS
_L = 16


@functools.lru_cache(maxsize=None)
def _make_sc_scores(N, V, D, K, T):
    n_per_w = N // _NW
    n_chunks = n_per_w // T
    n_groups = T // _L
    slots = 1 + K

    mesh = plsc.VectorSubcoreMesh(core_axis_name="c", subcore_axis_name="s")

    @functools.partial(
        pl.kernel,
        mesh=mesh,
        compiler_params=pltpu.CompilerParams(needs_layout_passes=False),
        out_type=jax.ShapeDtypeStruct((8, N), jnp.float32),
        scratch_types=[
            pltpu.VMEM((V * D,), jnp.float32),
            pltpu.VMEM((T * D,), jnp.float32),
            pltpu.VMEM((slots, T), jnp.int32),
            pltpu.VMEM((8, T), jnp.float32),
        ],
    )
    def sc_scores(ctx_hbm, idx_hbm, w_hbm, out_hbm, w_v, c_v, i_v, o_v):
        wid = lax.axis_index("s") * _NC + lax.axis_index("c")
        base = wid * n_per_w
        pltpu.sync_copy(w_hbm, w_v)

        lane = lax.iota(jnp.int32, _L)

        def chunk_body(ci, _):
            cbase = base + ci * T
            pltpu.sync_copy(ctx_hbm.at[pl.ds(cbase * D, T * D)], c_v)
            pltpu.sync_copy(idx_hbm.at[:, pl.ds(cbase, T)], i_v)

            def group_body(g, _g):
                goff = g * _L
                cidx0 = (goff + lane) * D
                widx0 = [i_v[k, pl.ds(goff, _L)] * D for k in range(slots)]
                zeros = jnp.zeros((_L,), jnp.float32)
                carry0 = tuple([cidx0] + widx0 + [zeros] * slots)

                def d_body(_, carry):
                    cidx = carry[0]
                    widx = carry[1:1 + slots]
                    accs = carry[1 + slots:]
                    cv = plsc.load_gather(c_v, [cidx])
                    new_accs = []
                    for k in range(slots):
                        wv = plsc.load_gather(w_v, [widx[k]])
                        new_accs.append(accs[k] + wv * cv)
                    return tuple(
                        [cidx + 1]
                        + [widx[k] + 1 for k in range(slots)]
                        + new_accs
                    )

                out = lax.fori_loop(0, D, d_body, carry0, unroll=4)
                accs = out[1 + slots:]
                o_v[0, pl.ds(goff, _L)] = accs[0]
                for k in range(1, slots):
                    o_v[k, pl.ds(goff, _L)] = -accs[k]
                for k in range(slots, 8):
                    o_v[k, pl.ds(goff, _L)] = zeros
                return 0

            lax.fori_loop(0, n_groups, group_body, 0)
            pltpu.sync_copy(o_v, out_hbm.at[:, pl.ds(cbase, T)])
            return 0

        lax.fori_loop(0, n_chunks, chunk_body, 0)

    return sc_scores


def _tc_reduce_body(s_ref, o_ref):
    s = s_ref[...]
    ls = jax.nn.log_sigmoid(s)
    row = lax.broadcasted_iota(jnp.int32, s.shape, 0)
    contrib = jnp.where(row < 6, ls, 0.0)
    o_ref[0, 0] = -jnp.sum(contrib)


def kernel(X, context, W, neg_samples):
    B, S = X.shape
    V, D = W.shape
    K = neg_samples.shape[-1]
    N = B * S

    T = 640
    assert N % (_NW * T) == 0

    ctx_flat = context.reshape(N * D)
    w_flat = W.reshape(V * D)
    x_row = X.reshape(1, N).astype(jnp.int32)
    neg_rows = neg_samples.reshape(N, K).T.astype(jnp.int32)
    idxT = jnp.concatenate([x_row, neg_rows], axis=0)

    scores = _make_sc_scores(N, V, D, K, T)(ctx_flat, idxT, w_flat)

    loss = pl.pallas_call(
        _tc_reduce_body,
        out_shape=jax.ShapeDtypeStruct((1, 1), jnp.float32),
        out_specs=pl.BlockSpec(memory_space=pltpu.SMEM),
    )(scores)
    return loss[0, 0]


# speedup vs baseline: 10.9227x; 2.7906x over previous
import functools

import jax
import jax.numpy as jnp
from jax import lax
from jax.experimental import pallas as pl
from jax.experimental.pallas import tpu as pltpu
from jax.experimental.pallas import tpu_sc as plsc

_NC = 2
_NS = 16
_NW = _NC * _NS
_L = 16


@functools.lru_cache(maxsize=None)
def _make_sc_scores(N, V, D, K, T, P):
    n_per_w = N // _NW
    n_chunks = n_per_w // T
    n_groups = T // _L
    slots = 1 + K

    mesh = plsc.VectorSubcoreMesh(core_axis_name="c", subcore_axis_name="s")

    @functools.partial(
        pl.kernel,
        mesh=mesh,
        compiler_params=pltpu.CompilerParams(needs_layout_passes=False),
        out_type=jax.ShapeDtypeStruct((8, N), jnp.float32),
        scratch_types=[
            pltpu.VMEM((V * P,), jnp.float32),
            pltpu.VMEM((T * P,), jnp.float32),
            pltpu.VMEM((slots, T), jnp.int32),
            pltpu.VMEM((8, T), jnp.float32),
        ],
    )
    def sc_scores(ctx_hbm, idx_hbm, w_hbm, out_hbm, w_v, c_v, i_v, o_v):
        wid = lax.axis_index("s") * _NC + lax.axis_index("c")
        base = wid * n_per_w
        pltpu.sync_copy(w_hbm, w_v)

        lane = lax.iota(jnp.int32, _L)

        def chunk_body(ci, _):
            cbase = base + ci * T
            pltpu.sync_copy(ctx_hbm.at[pl.ds(cbase * P, T * P)], c_v)
            pltpu.sync_copy(idx_hbm.at[:, pl.ds(cbase, T)], i_v)

            def group_body(g, _g):
                goff = g * _L
                cidx0 = (goff + lane) * P
                widx0 = [i_v[k, pl.ds(goff, _L)] * P for k in range(slots)]
                zeros = jnp.zeros((_L,), jnp.float32)
                carry0 = tuple([cidx0] + widx0 + [zeros] * slots)

                def d_body(_, carry):
                    cidx = carry[0]
                    widx = carry[1:1 + slots]
                    accs = carry[1 + slots:]
                    cv = plsc.load_gather(c_v, [cidx])
                    new_accs = []
                    for k in range(slots):
                        wv = plsc.load_gather(w_v, [widx[k]])
                        new_accs.append(accs[k] + wv * cv)
                    return tuple(
                        [cidx + 1]
                        + [widx[k] + 1 for k in range(slots)]
                        + new_accs
                    )

                out = lax.fori_loop(0, D, d_body, carry0, unroll=4)
                accs = out[1 + slots:]
                o_v[0, pl.ds(goff, _L)] = accs[0]
                for k in range(1, slots):
                    o_v[k, pl.ds(goff, _L)] = -accs[k]
                for k in range(slots, 8):
                    o_v[k, pl.ds(goff, _L)] = zeros
                return 0

            lax.fori_loop(0, n_groups, group_body, 0)
            pltpu.sync_copy(o_v, out_hbm.at[:, pl.ds(cbase, T)])
            return 0

        lax.fori_loop(0, n_chunks, chunk_body, 0)

    return sc_scores


def _tc_reduce_body(s_ref, o_ref):
    s = s_ref[...]
    ls = jax.nn.log_sigmoid(s)
    row = lax.broadcasted_iota(jnp.int32, s.shape, 0)
    contrib = jnp.where(row < 6, ls, 0.0)
    o_ref[0, 0] = -jnp.sum(contrib)


def kernel(X, context, W, neg_samples):
    B, S = X.shape
    V, D = W.shape
    K = neg_samples.shape[-1]
    N = B * S

    T = 640
    P = D + 1
    assert N % (_NW * T) == 0

    ctx_flat = jnp.pad(context.reshape(N, D), ((0, 0), (0, P - D))).reshape(N * P)
    w_flat = jnp.pad(W, ((0, 0), (0, P - D))).reshape(V * P)
    x_row = X.reshape(1, N).astype(jnp.int32)
    neg_rows = neg_samples.reshape(N, K).T.astype(jnp.int32)
    idxT = jnp.concatenate([x_row, neg_rows], axis=0)

    scores = _make_sc_scores(N, V, D, K, T, P)(ctx_flat, idxT, w_flat)

    loss = pl.pallas_call(
        _tc_reduce_body,
        out_shape=jax.ShapeDtypeStruct((1, 1), jnp.float32),
        out_specs=pl.BlockSpec(memory_space=pltpu.SMEM),
    )(scores)
    return loss[0, 0]


# speedup vs baseline: 12.8226x; 1.1739x over previous
import functools

import jax
import jax.numpy as jnp
from jax import lax
from jax.experimental import pallas as pl
from jax.experimental.pallas import tpu as pltpu
from jax.experimental.pallas import tpu_sc as plsc

_NC = 2
_NS = 16
_NW = _NC * _NS
_L = 16


@functools.lru_cache(maxsize=None)
def _make_sc_scores(N, V, D, K, T):
    n_per_w = N // _NW
    n_chunks = n_per_w // T
    n_groups = T // _L
    slots = 1 + K

    mesh = plsc.VectorSubcoreMesh(core_axis_name="c", subcore_axis_name="s")

    @functools.partial(
        pl.kernel,
        mesh=mesh,
        compiler_params=pltpu.CompilerParams(needs_layout_passes=False),
        out_type=jax.ShapeDtypeStruct((8, N), jnp.float32),
        scratch_types=[
            pltpu.VMEM((V * D,), jnp.float32),
            pltpu.VMEM((T * D,), jnp.float32),
            pltpu.VMEM((T,), jnp.int32),
            pltpu.VMEM((T * K,), jnp.int32),
            pltpu.VMEM((8, T), jnp.float32),
        ],
    )
    def sc_scores(ctx_hbm, x_hbm, neg_hbm, w_hbm, out_hbm, w_v, c_v, x_v,
                  n_v, o_v):
        wid = lax.axis_index("s") * _NC + lax.axis_index("c")
        base = wid * n_per_w
        pltpu.sync_copy(w_hbm, w_v)

        lane = lax.iota(jnp.int32, _L)

        def chunk_body(ci, _):
            cbase = base + ci * T
            pltpu.sync_copy(ctx_hbm.at[pl.ds(cbase * D, T * D)], c_v)
            pltpu.sync_copy(x_hbm.at[pl.ds(cbase, T)], x_v)
            pltpu.sync_copy(neg_hbm.at[pl.ds(cbase * K, T * K)], n_v)

            def group_body(g, _g):
                goff = g * _L
                nbase = (goff + lane) * K
                rowb = [x_v[pl.ds(goff, _L)] * D]
                for k in range(K):
                    rowb.append(plsc.load_gather(n_v, [nbase + k]) * D)
                crowb = (goff + lane) * D
                zeros = jnp.zeros((_L,), jnp.float32)
                carry0 = tuple([lane] + [zeros] * slots)

                def d_body(_, carry):
                    rot = carry[0]
                    accs = carry[1:]
                    cv = plsc.load_gather(c_v, [crowb + rot])
                    new_accs = []
                    for k in range(slots):
                        wv = plsc.load_gather(w_v, [rowb[k] + rot])
                        new_accs.append(accs[k] + wv * cv)
                    return tuple([(rot + 1) & (D - 1)] + new_accs)

                out = lax.fori_loop(0, D, d_body, carry0, unroll=4)
                accs = out[1:]
                o_v[0, pl.ds(goff, _L)] = accs[0]
                for k in range(1, slots):
                    o_v[k, pl.ds(goff, _L)] = -accs[k]
                for k in range(slots, 8):
                    o_v[k, pl.ds(goff, _L)] = zeros
                return 0

            lax.fori_loop(0, n_groups, group_body, 0)
            pltpu.sync_copy(o_v, out_hbm.at[:, pl.ds(cbase, T)])
            return 0

        lax.fori_loop(0, n_chunks, chunk_body, 0)

    return sc_scores


def _tc_reduce_body(s_ref, o_ref):
    s = s_ref[...]
    ls = jax.nn.log_sigmoid(s)
    row = lax.broadcasted_iota(jnp.int32, s.shape, 0)
    contrib = jnp.where(row < 6, ls, 0.0)
    o_ref[0, 0] = -jnp.sum(contrib)


def kernel(X, context, W, neg_samples):
    B, S = X.shape
    V, D = W.shape
    K = neg_samples.shape[-1]
    N = B * S

    T = 640
    assert N % (_NW * T) == 0
    assert D & (D - 1) == 0

    ctx_flat = context.reshape(N * D)
    w_flat = W.reshape(V * D)
    x_flat = X.reshape(N).astype(jnp.int32)
    neg_flat = neg_samples.reshape(N * K).astype(jnp.int32)

    scores = _make_sc_scores(N, V, D, K, T)(ctx_flat, x_flat, neg_flat, w_flat)

    loss = pl.pallas_call(
        _tc_reduce_body,
        out_shape=jax.ShapeDtypeStruct((1, 1), jnp.float32),
        out_specs=pl.BlockSpec(memory_space=pltpu.SMEM),
    )(scores)
    return loss[0, 0]
